# baseline (speedup 1.0000x reference)
import functools

import jax
import jax.numpy as jnp
from jax import lax
from jax.experimental import pallas as pl
from jax.experimental.pallas import tpu as pltpu
from jax.experimental.pallas import tpu_sc as plsc

Q = 1024
D = 256
N = 50000
NPAD = 51200
CHUNK = 128
NCHUNK = NPAD // CHUNK
NB = 2048
QB = 512
QB2 = 256
NB_KNN_KS = (10, 20, 100)
TEMP = 0.07
INV_T = 1.0 / TEMP
DELTA = 1.5
NUM_CLASSES = 1000
CPAD = 1024
NEG = -1e30
CAPC = 16
CAPH = 32
QPW = 32
NVC = NCHUNK // 16
STW = 1024
LMOFF = 512
MOFF = 992
FBOFF = 1008
SGRP = 8


def _gemm_body(x_ref, w_ref, labf_ref, s_ref, cm1_ref, cmx_ref, lm_ref):
    j = pl.program_id(0)
    s = lax.dot_general(
        x_ref[...], w_ref[...],
        dimension_numbers=(((1,), (1,)), ((), ())),
        preferred_element_type=jnp.float32,
    )
    col = j * NB + lax.broadcasted_iota(jnp.int32, (QB, NB), 1)
    s = jnp.where(col < N, s, NEG)
    s_ref[...] = s
    s3 = s.reshape(QB, NB // CHUNK, CHUNK)
    m1 = jnp.max(s3, axis=-1)
    eq = s3 == m1[:, :, None]
    eqf = eq.astype(jnp.float32).reshape(QB, NB)
    gsum = (lax.broadcasted_iota(jnp.int32, (NB, NB // CHUNK), 0) // CHUNK
            == lax.broadcasted_iota(jnp.int32, (NB, NB // CHUNK), 1)
            ).astype(jnp.float32)
    ceq = lax.dot_general(eqf, gsum, dimension_numbers=(((1,), (0,)), ((), ())),
                          preferred_element_type=jnp.float32)
    lm = lax.dot_general(eqf * labf_ref[...], gsum,
                         dimension_numbers=(((1,), (0,)), ((), ())),
                         precision=lax.Precision.HIGHEST,
                         preferred_element_type=jnp.float32)
    cm2 = jnp.max(jnp.where(eq, NEG, s3), axis=-1)
    cm1_ref[0, :, :] = m1
    cmx_ref[0, :, :] = jnp.where(ceq > 1.0, m1, cm2)
    lm_ref[0, :, :] = lm


def _sim_and_chunkstats(x, w_pad, labf):
    grid = (NPAD // NB, Q // QB)
    cm_spec = pl.BlockSpec((1, QB, NB // CHUNK), lambda j, i: (j, i, 0))
    cm_shape = jax.ShapeDtypeStruct((NPAD // NB, Q, NB // CHUNK), jnp.float32)
    return pl.pallas_call(
        _gemm_body,
        grid=grid,
        in_specs=[
            pl.BlockSpec((QB, D), lambda j, i: (i, 0)),
            pl.BlockSpec((NB, D), lambda j, i: (j, 0)),
            pl.BlockSpec((1, NB), lambda j, i: (0, j)),
        ],
        out_specs=[
            pl.BlockSpec((QB, NB), lambda j, i: (i, j)),
            cm_spec, cm_spec, cm_spec,
        ],
        out_shape=[
            jax.ShapeDtypeStruct((Q, NPAD), jnp.float32),
            cm_shape, cm_shape, cm_shape,
        ],
    )(x, w_pad, labf)


def _stats_body(cm1_ref, cmx_ref, lm_ref, st_ref):
    cm1 = cm1_ref[...]
    m = jnp.max(jnp.max(cm1, axis=0), axis=1)
    tau = m - DELTA
    sel = cm1 >= tau[None, :, None]
    worst = jnp.max(jnp.max(jnp.where(sel, cmx_ref[...], NEG), axis=0),
                    axis=1)
    fb = (worst >= tau).astype(jnp.float32)
    for j in range(NVC):
        st_ref[:, pl.ds(j * 16, 16)] = cm1[j]
        st_ref[:, pl.ds(LMOFF + j * 16, 16)] = lm_ref[j, :, :]
    st_ref[:, pl.ds(MOFF, 16)] = jnp.broadcast_to(m[:, None], (QB2, 16))
    st_ref[:, pl.ds(FBOFF, 16)] = jnp.broadcast_to(fb[:, None], (QB2, 16))


def _stats(cm1, cmx, lm):
    grid = (Q // QB2,)
    in_spec = pl.BlockSpec((NPAD // NB, QB2, NB // CHUNK), lambda i: (0, i, 0))
    return pl.pallas_call(
        _stats_body,
        grid=grid,
        in_specs=[in_spec, in_spec, in_spec],
        out_specs=pl.BlockSpec((QB2, STW), lambda i: (i, 0)),
        out_shape=jax.ShapeDtypeStruct((Q, STW), jnp.float32),
    )(cm1, cmx, lm)


def _sc_body(s_chunks, st_hbm, lab_chunks, out_hbm,
             st_loc, cids, lidxf, sidxf, cand,
             labc, hvals, hlabsf, votesq, semg1, semg2, semv0, semv1):
    wid = lax.axis_index("s") * 2 + lax.axis_index("c")
    q0 = wid * QPW
    iota = lax.iota(jnp.int32, 16)
    zf = jnp.zeros((16,), jnp.float32)
    zi = jnp.zeros((16,), jnp.int32)
    negv = jnp.full((16,), NEG, jnp.float32)
    semv = (semv0, semv1)

    for t in range(3):
        hlabsf[pl.ds(t * 16, 16)] = zf

    def _vote_waits(ql, slot):
        for g in range(3):
            pltpu.make_async_copy(votesq.at[slot, g],
                                  out_hbm.at[g, pl.ds(q0 + ql, 1), :],
                                  semv[slot]).wait()

    def _handle(ql, slot, i):
        q = q0 + ql
        qs = ql - (ql // SGRP) * SGRP
        m16 = st_loc[qs, pl.ds(MOFF, 16)]
        tau16 = m16 - DELTA
        fb = jnp.max(st_loc[qs, pl.ds(FBOFF, 16)]) > 0.5

        for t in range(3):
            hvals[pl.ds(t * 16, 16)] = negv
        for t in range(2):
            cids[pl.ds(t * 16, 16)] = jnp.full((16,), NCHUNK - 1, jnp.int32)

        def _csel(j, cnt):
            v = st_loc[qs, pl.ds(j * 16, 16)]
            mask = v >= tau16
            off = jnp.minimum(cnt, CAPC)
            plsc.store_compressed(cids.at[pl.ds(off, 16)], iota + j * 16,
                                  mask=mask)
            plsc.store_compressed(hvals.at[pl.ds(off, 16)], v, mask=mask)
            plsc.store_compressed(hlabsf.at[pl.ds(off, 16)],
                                  st_loc[qs, pl.ds(LMOFF + j * 16, 16)],
                                  mask=mask)
            return cnt + jnp.sum(mask.astype(jnp.int32))
        cnt = lax.fori_loop(0, NVC, _csel, 0)

        @pl.when(fb)
        def _():
            ncl = jnp.minimum(cnt, CAPC)
            cv = cids[pl.ds(0, 16)]
            lidxf[...] = cv
            sidxf[...] = cv + q * NCHUNK
            pltpu.async_copy(s_chunks.at[sidxf], cand, semg1)
            pltpu.async_copy(lab_chunks.at[lidxf], labc, semg2)
            pltpu.make_async_copy(s_chunks.at[sidxf], cand, semg1).wait()
            pltpu.make_async_copy(lab_chunks.at[lidxf], labc, semg2).wait()
            for t in range(3):
                hvals[pl.ds(t * 16, 16)] = negv

            def _hsel(j, hcnt):
                for u in range(CHUNK // 16):
                    v = cand[j, pl.ds(u * 16, 16)]
                    mask = v >= tau16
                    hoff = jnp.minimum(hcnt, CAPH)
                    plsc.store_compressed(hvals.at[pl.ds(hoff, 16)], v,
                                          mask=mask)
                    plsc.store_compressed(
                        hlabsf.at[pl.ds(hoff, 16)],
                        labc[j, pl.ds(u * 16, 16)].astype(jnp.float32),
                        mask=mask)
                    hcnt = hcnt + jnp.sum(mask.astype(jnp.int32))
                return hcnt
            lax.fori_loop(0, ncl, _hsel, 0)

        v0 = hvals[pl.ds(0, 16)]
        v1 = hvals[pl.ds(16, 16)]
        e0 = jnp.exp((v0 - m16) * INV_T)
        e1 = jnp.exp((v1 - m16) * INV_T)
        den = jnp.sum(e0) + jnp.sum(e1)
        w0 = e0 / den
        w1 = e1 / den

        r0 = zi
        r1 = zi
        for src in (v0, v1):
            for ln in range(16):
                sv = src[ln]
                r0 = r0 + (sv > v0).astype(jnp.int32)
                r1 = r1 + (sv > v1).astype(jnp.int32)

        @pl.when(i > 0)
        def _():
            _vote_waits(ql - 2, slot)
        for g in range(3):
            for u in range(CPAD // 16):
                votesq[slot, g, 0, pl.ds(u * 16, 16)] = zf

        l0 = hlabsf[pl.ds(0, 16)].astype(jnp.int32)
        l1 = hlabsf[pl.ds(16, 16)].astype(jnp.int32)
        sv16 = jnp.full((16,), slot, jnp.int32)
        for g, kk in enumerate(NB_KNN_KS):
            gi = jnp.full((16,), g, jnp.int32)
            plsc.addupdate_scatter(votesq, [sv16, gi, zi, l0],
                                   jnp.where(r0 < kk, w0, 0.0))
            plsc.addupdate_scatter(votesq, [sv16, gi, zi, l1],
                                   jnp.where(r1 < kk, w1, 0.0))
        for g in range(3):
            pltpu.async_copy(votesq.at[slot, g],
                             out_hbm.at[g, pl.ds(q0 + ql, 1), :], semv[slot])

    def _pair(i, carry):
        @pl.when(i - (i // (SGRP // 2)) * (SGRP // 2) == 0)
        def _():
            base = pl.multiple_of(q0 + (i // (SGRP // 2)) * SGRP, SGRP)
            pltpu.sync_copy(st_hbm.at[pl.ds(base, SGRP), :], st_loc)
        _handle(2 * i, 0, i)
        _handle(2 * i + 1, 1, i)
        return carry
    lax.fori_loop(0, QPW // 2, _pair, 0)

    _vote_waits(QPW - 2, 0)
    _vote_waits(QPW - 1, 1)


@functools.cache
def _get_sc_vote():
  return pl.kernel(
    _sc_body,
    out_type=jax.ShapeDtypeStruct((3, Q, CPAD), jnp.float32),
    mesh=plsc.VectorSubcoreMesh(core_axis_name="c", subcore_axis_name="s"),
    compiler_params=pltpu.CompilerParams(needs_layout_passes=False),
    scratch_types=[
        pltpu.VMEM((SGRP, STW), jnp.float32),
        pltpu.VMEM((CAPC + 16,), jnp.int32),
        pltpu.VMEM((CAPC,), jnp.int32),
        pltpu.VMEM((CAPC,), jnp.int32),
        pltpu.VMEM((CAPC, CHUNK), jnp.float32),
        pltpu.VMEM((CAPC, CHUNK), jnp.int32),
        pltpu.VMEM((CAPH + 16,), jnp.float32),
        pltpu.VMEM((CAPH + 16,), jnp.float32),
        pltpu.VMEM((2, 3, 1, CPAD), jnp.float32),
        pltpu.SemaphoreType.DMA,
        pltpu.SemaphoreType.DMA,
        pltpu.SemaphoreType.DMA,
        pltpu.SemaphoreType.DMA,
    ],
  )


@jax.jit
def _knn(features_rank, train_features, train_labels):
    w_pad = jnp.zeros((NPAD, D), jnp.float32).at[:N].set(train_features)
    labf = (jnp.zeros((NPAD,), jnp.float32)
            .at[:N].set(train_labels.astype(jnp.float32)).reshape(1, NPAD))
    s, cm1, cmx, lm = _sim_and_chunkstats(features_rank, w_pad, labf)
    st = _stats(cm1, cmx, lm)
    s_chunks = s.reshape(Q * NCHUNK, CHUNK)
    lab_chunks = (jnp.zeros((NPAD,), jnp.int32).at[:N].set(train_labels)
                  .reshape(NCHUNK, CHUNK))
    out = _get_sc_vote()(s_chunks, st, lab_chunks)
    return out[:, :, :NUM_CLASSES]


def kernel(features_rank, train_features, train_labels):
    return _knn(features_rank, train_features, train_labels)


# speedup vs baseline: 1.3093x; 1.3093x over previous
import functools

import jax
import jax.numpy as jnp
from jax import lax
from jax.experimental import pallas as pl
from jax.experimental.pallas import tpu as pltpu
from jax.experimental.pallas import tpu_sc as plsc

Q = 1024
D = 256
N = 50000
NPAD = 51200
CHUNK = 128
NCHUNK = NPAD // CHUNK
NB = 2048
QB = 512
QB2 = 256
NB_KNN_KS = (10, 20, 100)
TEMP = 0.07
INV_T = 1.0 / TEMP
DELTA = 1.5
NUM_CLASSES = 1000
CPAD = 1024
NEG = -1e30
CAPC = 16
CAPH = 32
QPW = 32
NVC = NCHUNK // 16
STW = 1024
LMOFF = 512
MOFF = 992
FBOFF = 1008
SGRP = 8


def _gemm_body(x_ref, w_ref, labf_ref, s_ref, cm1_ref, cmx_ref, lm_ref):
    j = pl.program_id(0)
    s = lax.dot_general(
        x_ref[...], w_ref[...],
        dimension_numbers=(((1,), (1,)), ((), ())),
        preferred_element_type=jnp.float32,
    )
    col = j * NB + lax.broadcasted_iota(jnp.int32, (QB, NB), 1)
    s = jnp.where(col < N, s, NEG)
    s_ref[...] = s
    s3 = s.reshape(QB, NB // CHUNK, CHUNK)
    m1 = jnp.max(s3, axis=-1)
    eq = s3 == m1[:, :, None]
    eqf = eq.astype(jnp.float32).reshape(QB, NB)
    gsum = (lax.broadcasted_iota(jnp.int32, (NB, NB // CHUNK), 0) // CHUNK
            == lax.broadcasted_iota(jnp.int32, (NB, NB // CHUNK), 1)
            ).astype(jnp.float32)
    ceq = lax.dot_general(eqf, gsum, dimension_numbers=(((1,), (0,)), ((), ())),
                          preferred_element_type=jnp.float32)
    labf = labf_ref[...]
    labhi = jnp.floor(labf * (1.0 / 32.0))
    lablo = labf - labhi * 32.0
    dn = (((1,), (0,)), ((), ()))
    lm = (lax.dot_general(eqf * labhi, gsum, dimension_numbers=dn,
                          preferred_element_type=jnp.float32) * 32.0
          + lax.dot_general(eqf * lablo, gsum, dimension_numbers=dn,
                            preferred_element_type=jnp.float32))
    cm2 = jnp.max(jnp.where(eq, NEG, s3), axis=-1)
    cm1_ref[0, :, :] = m1
    cmx_ref[0, :, :] = jnp.where(ceq > 1.0, m1, cm2)
    lm_ref[0, :, :] = lm


def _sim_and_chunkstats(x, w_pad, labf):
    grid = (NPAD // NB, Q // QB)
    cm_spec = pl.BlockSpec((1, QB, NB // CHUNK), lambda j, i: (j, i, 0))
    cm_shape = jax.ShapeDtypeStruct((NPAD // NB, Q, NB // CHUNK), jnp.float32)
    return pl.pallas_call(
        _gemm_body,
        grid=grid,
        in_specs=[
            pl.BlockSpec((QB, D), lambda j, i: (i, 0)),
            pl.BlockSpec((NB, D), lambda j, i: (j, 0)),
            pl.BlockSpec((1, NB), lambda j, i: (0, j)),
        ],
        out_specs=[
            pl.BlockSpec((QB, NB), lambda j, i: (i, j)),
            cm_spec, cm_spec, cm_spec,
        ],
        out_shape=[
            jax.ShapeDtypeStruct((Q, NPAD), jnp.float32),
            cm_shape, cm_shape, cm_shape,
        ],
    )(x, w_pad, labf)


def _stats_body(cm1_ref, cmx_ref, lm_ref, st_ref):
    cm1 = cm1_ref[...]
    m = jnp.max(jnp.max(cm1, axis=0), axis=1)
    tau = m - DELTA
    sel = cm1 >= tau[None, :, None]
    worst = jnp.max(jnp.max(jnp.where(sel, cmx_ref[...], NEG), axis=0),
                    axis=1)
    fb = (worst >= tau).astype(jnp.float32)
    for j in range(NVC):
        st_ref[:, pl.ds(j * 16, 16)] = cm1[j]
        st_ref[:, pl.ds(LMOFF + j * 16, 16)] = lm_ref[j, :, :]
    st_ref[:, pl.ds(MOFF, 16)] = jnp.broadcast_to(m[:, None], (QB2, 16))
    st_ref[:, pl.ds(FBOFF, 16)] = jnp.broadcast_to(fb[:, None], (QB2, 16))


def _stats(cm1, cmx, lm):
    grid = (Q // QB2,)
    in_spec = pl.BlockSpec((NPAD // NB, QB2, NB // CHUNK), lambda i: (0, i, 0))
    return pl.pallas_call(
        _stats_body,
        grid=grid,
        in_specs=[in_spec, in_spec, in_spec],
        out_specs=pl.BlockSpec((QB2, STW), lambda i: (i, 0)),
        out_shape=jax.ShapeDtypeStruct((Q, STW), jnp.float32),
    )(cm1, cmx, lm)


def _sc_body(s_chunks, st_hbm, lab_chunks, out_hbm,
             st_loc, cids, lidxf, sidxf, cand,
             labc, hvals, hlabsf, votesq, semg1, semg2, semv0, semv1):
    wid = lax.axis_index("s") * 2 + lax.axis_index("c")
    q0 = wid * QPW
    iota = lax.iota(jnp.int32, 16)
    zf = jnp.zeros((16,), jnp.float32)
    zi = jnp.zeros((16,), jnp.int32)
    negv = jnp.full((16,), NEG, jnp.float32)
    semv = (semv0, semv1)

    for t in range(3):
        hlabsf[pl.ds(t * 16, 16)] = zf

    def _vote_waits(ql, slot):
        for g in range(3):
            pltpu.make_async_copy(votesq.at[slot, g],
                                  out_hbm.at[g, pl.ds(q0 + ql, 1), :],
                                  semv[slot]).wait()

    def _handle(ql, slot, i):
        q = q0 + ql
        qs = ql - (ql // SGRP) * SGRP
        m16 = st_loc[qs, pl.ds(MOFF, 16)]
        tau16 = m16 - DELTA
        fb = jnp.max(st_loc[qs, pl.ds(FBOFF, 16)]) > 0.5

        for t in range(3):
            hvals[pl.ds(t * 16, 16)] = negv
        for t in range(2):
            cids[pl.ds(t * 16, 16)] = jnp.full((16,), NCHUNK - 1, jnp.int32)

        def _csel(j, cnt):
            v = st_loc[qs, pl.ds(j * 16, 16)]
            mask = v >= tau16
            off = jnp.minimum(cnt, CAPC)
            plsc.store_compressed(cids.at[pl.ds(off, 16)], iota + j * 16,
                                  mask=mask)
            plsc.store_compressed(hvals.at[pl.ds(off, 16)], v, mask=mask)
            plsc.store_compressed(hlabsf.at[pl.ds(off, 16)],
                                  st_loc[qs, pl.ds(LMOFF + j * 16, 16)],
                                  mask=mask)
            return cnt + jnp.sum(mask.astype(jnp.int32))
        cnt = lax.fori_loop(0, NVC, _csel, 0)

        @pl.when(fb)
        def _():
            ncl = jnp.minimum(cnt, CAPC)
            cv = cids[pl.ds(0, 16)]
            lidxf[...] = cv
            sidxf[...] = cv + q * NCHUNK
            pltpu.async_copy(s_chunks.at[sidxf], cand, semg1)
            pltpu.async_copy(lab_chunks.at[lidxf], labc, semg2)
            pltpu.make_async_copy(s_chunks.at[sidxf], cand, semg1).wait()
            pltpu.make_async_copy(lab_chunks.at[lidxf], labc, semg2).wait()
            for t in range(3):
                hvals[pl.ds(t * 16, 16)] = negv

            def _hsel(j, hcnt):
                for u in range(CHUNK // 16):
                    v = cand[j, pl.ds(u * 16, 16)]
                    mask = v >= tau16
                    hoff = jnp.minimum(hcnt, CAPH)
                    plsc.store_compressed(hvals.at[pl.ds(hoff, 16)], v,
                                          mask=mask)
                    plsc.store_compressed(
                        hlabsf.at[pl.ds(hoff, 16)],
                        labc[j, pl.ds(u * 16, 16)].astype(jnp.float32),
                        mask=mask)
                    hcnt = hcnt + jnp.sum(mask.astype(jnp.int32))
                return hcnt
            lax.fori_loop(0, ncl, _hsel, 0)

        v0 = hvals[pl.ds(0, 16)]
        v1 = hvals[pl.ds(16, 16)]
        e0 = jnp.exp((v0 - m16) * INV_T)
        e1 = jnp.exp((v1 - m16) * INV_T)
        den = jnp.sum(e0) + jnp.sum(e1)
        w0 = e0 / den
        w1 = e1 / den

        r0 = zi
        r1 = zi
        for src in (v0, v1):
            for ln in range(16):
                sv = src[ln]
                r0 = r0 + (sv > v0).astype(jnp.int32)
                r1 = r1 + (sv > v1).astype(jnp.int32)

        @pl.when(i > 0)
        def _():
            _vote_waits(ql - 2, slot)
        for g in range(3):
            for u in range(CPAD // 16):
                votesq[slot, g, 0, pl.ds(u * 16, 16)] = zf

        l0 = hlabsf[pl.ds(0, 16)].astype(jnp.int32)
        l1 = hlabsf[pl.ds(16, 16)].astype(jnp.int32)
        sv16 = jnp.full((16,), slot, jnp.int32)
        for g, kk in enumerate(NB_KNN_KS):
            gi = jnp.full((16,), g, jnp.int32)
            plsc.addupdate_scatter(votesq, [sv16, gi, zi, l0],
                                   jnp.where(r0 < kk, w0, 0.0))
            plsc.addupdate_scatter(votesq, [sv16, gi, zi, l1],
                                   jnp.where(r1 < kk, w1, 0.0))
        for g in range(3):
            pltpu.async_copy(votesq.at[slot, g],
                             out_hbm.at[g, pl.ds(q0 + ql, 1), :], semv[slot])

    def _pair(i, carry):
        @pl.when(i - (i // (SGRP // 2)) * (SGRP // 2) == 0)
        def _():
            base = pl.multiple_of(q0 + (i // (SGRP // 2)) * SGRP, SGRP)
            pltpu.sync_copy(st_hbm.at[pl.ds(base, SGRP), :], st_loc)
        _handle(2 * i, 0, i)
        _handle(2 * i + 1, 1, i)
        return carry
    lax.fori_loop(0, QPW // 2, _pair, 0)

    _vote_waits(QPW - 2, 0)
    _vote_waits(QPW - 1, 1)


@functools.cache
def _get_sc_vote():
  return pl.kernel(
    _sc_body,
    out_type=jax.ShapeDtypeStruct((3, Q, CPAD), jnp.float32),
    mesh=plsc.VectorSubcoreMesh(core_axis_name="c", subcore_axis_name="s"),
    compiler_params=pltpu.CompilerParams(needs_layout_passes=False),
    scratch_types=[
        pltpu.VMEM((SGRP, STW), jnp.float32),
        pltpu.VMEM((CAPC + 16,), jnp.int32),
        pltpu.VMEM((CAPC,), jnp.int32),
        pltpu.VMEM((CAPC,), jnp.int32),
        pltpu.VMEM((CAPC, CHUNK), jnp.float32),
        pltpu.VMEM((CAPC, CHUNK), jnp.int32),
        pltpu.VMEM((CAPH + 16,), jnp.float32),
        pltpu.VMEM((CAPH + 16,), jnp.float32),
        pltpu.VMEM((2, 3, 1, CPAD), jnp.float32),
        pltpu.SemaphoreType.DMA,
        pltpu.SemaphoreType.DMA,
        pltpu.SemaphoreType.DMA,
        pltpu.SemaphoreType.DMA,
    ],
  )


@jax.jit
def _knn(features_rank, train_features, train_labels):
    w_pad = jnp.zeros((NPAD, D), jnp.float32).at[:N].set(train_features)
    labf = (jnp.zeros((NPAD,), jnp.float32)
            .at[:N].set(train_labels.astype(jnp.float32)).reshape(1, NPAD))
    s, cm1, cmx, lm = _sim_and_chunkstats(features_rank, w_pad, labf)
    st = _stats(cm1, cmx, lm)
    s_chunks = s.reshape(Q * NCHUNK, CHUNK)
    lab_chunks = (jnp.zeros((NPAD,), jnp.int32).at[:N].set(train_labels)
                  .reshape(NCHUNK, CHUNK))
    out = _get_sc_vote()(s_chunks, st, lab_chunks)
    return out[:, :, :NUM_CLASSES]


def kernel(features_rank, train_features, train_labels):
    return _knn(features_rank, train_features, train_labels)
